# baseline (speedup 1.0000x reference)
import functools

import jax
import jax.numpy as jnp
from jax import lax
from jax.experimental import pallas as pl
from jax.experimental.pallas import tpu as pltpu

_BLK = 1024


def _tc_embed(idx3, tab_t):
    nb = idx3.shape[0]
    two_d, vocab = tab_t.shape
    embed_dim = two_d // 2
    rows = _BLK * embed_dim // 128

    def body(idx_ref, tab_ref, out_ref):
        idx = idx_ref[0]
        vio = lax.broadcasted_iota(jnp.int32, (vocab, 128), 0)
        for s in range(8):
            oh = (idx[s:s + 1, :] == vio).astype(jnp.bfloat16)
            r = lax.dot_general(tab_ref[...], oh, (((1,), (0,)), ((), ())),
                                preferred_element_type=jnp.float32)
            t = lax.transpose(r[:embed_dim] + r[embed_dim:], (1, 0))
            out_ref[pl.ds(128 * (s % 2), 128),
                    pl.ds(embed_dim * (s // 2), embed_dim)] = t

    return pl.pallas_call(
        body,
        grid=(nb,),
        in_specs=[
            pl.BlockSpec((1, 8, 128), lambda i: (i, 0, 0)),
            pl.BlockSpec((two_d, vocab), lambda i: (0, 0)),
        ],
        out_specs=pl.BlockSpec((rows, 128), lambda i: (i, 0)),
        out_shape=jax.ShapeDtypeStruct((nb * rows, 128), jnp.float32),
    )(idx3, tab_t)


def kernel(indices, table):
    batch, hist = indices.shape
    vocab, embed_dim = table.shape
    n = batch * hist
    nb = n // _BLK
    th = table.astype(jnp.bfloat16)
    tl = (table - th.astype(jnp.float32)).astype(jnp.bfloat16)
    tab_t = jnp.concatenate([th, tl], axis=1).T
    idx3 = (indices.reshape(nb, _BLK // 4, 4)
            .transpose(0, 2, 1).reshape(nb, 8, 128))
    out = _tc_embed(idx3, tab_t)
    return out.reshape(batch, hist, embed_dim)
